# baseline (speedup 1.0000x reference)
import functools

import jax
import jax.numpy as jnp
from jax import lax
from jax.experimental import pallas as pl
from jax.experimental.pallas import tpu as pltpu
from jax.experimental.pallas import tpu_sc as plsc

_BATCH = 16384
_EMBED = 64
_NW = 32
_BPW = _BATCH // _NW
_CHUNK = 128
_NCHUNK = _BPW // _CHUNK


def _make_gather():
    mesh = plsc.VectorSubcoreMesh(core_axis_name="c", subcore_axis_name="s")

    @functools.partial(
        pl.kernel,
        mesh=mesh,
        out_type=jax.ShapeDtypeStruct((_BATCH // 2, 2 * _EMBED), jnp.float32),
        scratch_types=[
            pltpu.VMEM((_NCHUNK, _CHUNK), jnp.int32),
            pltpu.VMEM((_NCHUNK, _CHUNK), jnp.int32),
            pltpu.VMEM((_BPW, 2 * _EMBED), jnp.float32),
            pltpu.VMEM((_BPW // 2, 2 * _EMBED), jnp.float32),
            pltpu.SemaphoreType.DMA,
        ],
        compiler_params=pltpu.CompilerParams(
            use_tc_tiling_on_sc=True, needs_layout_passes=False
        ),
    )
    def gather_kernel(idx_hbm, t2_hbm, out_hbm, xv, par, pairs, stage, sem):
        wid = lax.axis_index("s") * 2 + lax.axis_index("c")
        base = wid * _BPW
        for j in range(_NCHUNK):
            pltpu.sync_copy(idx_hbm.at[pl.ds(base + j * _CHUNK, _CHUNK)], xv.at[j])
        for j in range(_NCHUNK):
            for k in range(_CHUNK // 16):
                v = xv[j, pl.ds(k * 16, 16)]
                par[j, pl.ds(k * 16, 16)] = (v & 1) * _EMBED
                xv[j, pl.ds(k * 16, 16)] = lax.shift_right_logical(v, 1)
        copies = [
            pltpu.async_copy(
                t2_hbm.at[xv.at[j]],
                pairs.at[pl.ds(j * _CHUNK, _CHUNK)],
                sem,
            )
            for j in range(_NCHUNK)
        ]
        for c in copies:
            c.wait()

        iota = lax.iota(jnp.int32, 16)
        for g in range(_BPW // 16):
            iv = g * 16 + iota
            pv = par[g // 8, pl.ds((g % 8) * 16, 16)]
            rows2 = lax.shift_right_logical(iv, 1)
            cols2 = (iv & 1) * _EMBED

            def body(e, _):
                vals = plsc.load_gather(pairs, [iv, pv + e])
                plsc.store_scatter(stage, [rows2, cols2 + e], vals)
                return ()

            lax.fori_loop(0, _EMBED, body, ())

        pltpu.sync_copy(stage, out_hbm.at[pl.ds(wid * (_BPW // 2), _BPW // 2)])

    return gather_kernel


_gather = _make_gather()


@jax.jit
def kernel(X, embedding):
    t2 = jnp.reshape(embedding, (500000, 128))
    out2 = _gather(X.astype(jnp.int32), t2)
    return jnp.reshape(out2, (_BATCH, _EMBED))


# speedup vs baseline: 1.6597x; 1.6597x over previous
import functools

import jax
import jax.numpy as jnp
from jax import lax
from jax.experimental import pallas as pl
from jax.experimental.pallas import tpu as pltpu
from jax.experimental.pallas import tpu_sc as plsc

_BATCH = 16384
_EMBED = 64
_NW = 32
_BPW = _BATCH // _NW
_CHUNK = 32
_NCHUNK = _BPW // _CHUNK


def _make_gather():
    mesh = plsc.VectorSubcoreMesh(core_axis_name="c", subcore_axis_name="s")

    @functools.partial(
        pl.kernel,
        mesh=mesh,
        out_type=jax.ShapeDtypeStruct((_BATCH, _EMBED), jnp.float32),
        scratch_types=[
            pltpu.VMEM((4, 128), jnp.int32),
            pltpu.VMEM((_CHUNK * 8, _EMBED), jnp.float32),
            pltpu.VMEM((_BPW, _EMBED), jnp.float32),
            pltpu.SemaphoreType.DMA,
        ],
        compiler_params=pltpu.CompilerParams(
            use_tc_tiling_on_sc=True, needs_layout_passes=False
        ),
    )
    def gather_kernel(idx_hbm, table_hbm, out_hbm, xv, bands, outv, sem):
        wid = lax.axis_index("s") * 2 + lax.axis_index("c")
        base = wid * _BPW
        for j in range(4):
            pltpu.sync_copy(idx_hbm.at[pl.ds(base + j * 128, 128)], xv.at[j])

        def chunk_body(g, _):
            blks = [
                xv[g >> 2, pl.ds((g & 3) * _CHUNK + t * 16, 16)]
                for t in range(_CHUNK // 16)
            ]
            copies = []
            for k in range(_CHUNK):
                v = blks[k // 16][k % 16]
                vb = pl.multiple_of((v >> 3) * 8, 8)
                copies.append(
                    pltpu.async_copy(
                        table_hbm.at[pl.ds(vb, 8)],
                        bands.at[pl.ds(k * 8, 8)],
                        sem,
                    )
                )
            for c in copies:
                c.wait()
            for k in range(_CHUNK):
                v = blks[k // 16][k % 16]
                r = k * 8 + (v & 7)
                i_loc = g * _CHUNK + k
                for eb in range(_EMBED // 16):
                    outv[i_loc, pl.ds(eb * 16, 16)] = bands[r, pl.ds(eb * 16, 16)]
            return ()

        lax.fori_loop(0, _NCHUNK, chunk_body, ())
        pltpu.sync_copy(outv, out_hbm.at[pl.ds(base, _BPW)])

    return gather_kernel

_gather = _make_gather()


@jax.jit
def kernel(X, embedding):
    return _gather(X.astype(jnp.int32), embedding)


# speedup vs baseline: 2.3607x; 1.4223x over previous
import functools

import jax
import jax.numpy as jnp
from jax import lax
from jax.experimental import pallas as pl
from jax.experimental.pallas import tpu as pltpu
from jax.experimental.pallas import tpu_sc as plsc

_BATCH = 16384
_EMBED = 64
_NW = 32
_BPW = _BATCH // _NW
_CHUNK = 32
_NCHUNK = _BPW // _CHUNK


def _make_gather():
    mesh = plsc.VectorSubcoreMesh(core_axis_name="c", subcore_axis_name="s")

    @functools.partial(
        pl.kernel,
        mesh=mesh,
        out_type=jax.ShapeDtypeStruct((_BATCH, _EMBED), jnp.float32),
        scratch_types=[
            pltpu.VMEM((4, 128), jnp.int32),
            pltpu.VMEM((_CHUNK, 8, _EMBED), jnp.float32),
            pltpu.VMEM((_BPW, _EMBED), jnp.float32),
            pltpu.SemaphoreType.DMA,
        ],
        compiler_params=pltpu.CompilerParams(
            use_tc_tiling_on_sc=True, needs_layout_passes=False
        ),
    )
    def gather_kernel(idx_hbm, table_hbm, out_hbm, xv, bands3, outv, sem):

        wid = lax.axis_index("s") * 2 + lax.axis_index("c")
        base = wid * _BPW
        for j in range(4):
            pltpu.sync_copy(idx_hbm.at[pl.ds(base + j * 128, 128)], xv.at[j])

        def chunk_body(g, _):
            blks = [
                xv[g >> 2, pl.ds((g & 3) * _CHUNK + t * 16, 16)]
                for t in range(_CHUNK // 16)
            ]
            copies = []
            for k in range(_CHUNK):
                v = blks[k // 16][k % 16]
                copies.append(
                    pltpu.async_copy(
                        table_hbm.at[pl.ds(v >> 3, 1)],
                        bands3.at[pl.ds(k, 1)],
                        sem,
                    )
                )
            for c in copies:
                c.wait()
            for k in range(_CHUNK):
                v = blks[k // 16][k % 16]
                r = v & 7
                i_loc = g * _CHUNK + k
                for eb in range(_EMBED // 16):
                    outv[i_loc, pl.ds(eb * 16, 16)] = bands3[k, r, pl.ds(eb * 16, 16)]
            return ()

        lax.fori_loop(0, _NCHUNK, chunk_body, ())
        pltpu.sync_copy(outv, out_hbm.at[pl.ds(base, _BPW)])

    return gather_kernel

_gather = _make_gather()


@jax.jit
def kernel(X, embedding):
    t3 = jnp.reshape(embedding, (125000, 8, _EMBED))
    return _gather(X.astype(jnp.int32), t3)


# speedup vs baseline: 2.4632x; 1.0434x over previous
import functools

import jax
import jax.numpy as jnp
from jax import lax
from jax.experimental import pallas as pl
from jax.experimental.pallas import tpu as pltpu
from jax.experimental.pallas import tpu_sc as plsc

_BATCH = 16384
_EMBED = 64
_NW = 32
_BPW = _BATCH // _NW
_CHUNK = 16
_NCHUNK = _BPW // _CHUNK


def _make_gather():
    mesh = plsc.VectorSubcoreMesh(core_axis_name="c", subcore_axis_name="s")

    @functools.partial(
        pl.kernel,
        mesh=mesh,
        out_type=jax.ShapeDtypeStruct((_BATCH, _EMBED), jnp.float32),
        scratch_types=[
            pltpu.VMEM((4, 128), jnp.int32),
            pltpu.VMEM((2 * _CHUNK, 8, _EMBED), jnp.float32),
            pltpu.VMEM((_BPW, _EMBED), jnp.float32),
            pltpu.SemaphoreType.DMA,
        ],
        compiler_params=pltpu.CompilerParams(
            use_tc_tiling_on_sc=True, needs_layout_passes=False
        ),
    )
    def gather_kernel(idx_hbm, table_hbm, out_hbm, xv, bands2, outv, sem):

        wid = lax.axis_index("s") * 2 + lax.axis_index("c")
        base = wid * _BPW
        for j in range(4):
            pltpu.sync_copy(idx_hbm.at[pl.ds(base + j * 128, 128)], xv.at[j])

        def enqueue(c, buf):
            blk = xv[c >> 3, pl.ds((c & 7) * 16, 16)]
            for k in range(_CHUNK):
                v = blk[k]
                pltpu.async_copy(
                    table_hbm.at[pl.ds(v >> 3, 1)],
                    bands2.at[pl.ds(buf * _CHUNK + k, 1)],
                    sem,
                )

        enqueue(0, 0)

        def chunk_body(g, _):
            nxt = g + 1

            @pl.when(nxt < _NCHUNK)
            def _():
                enqueue(nxt, nxt & 1)

            for k in range(_CHUNK):
                pltpu.make_async_copy(
                    table_hbm.at[pl.ds(0, 1)],
                    bands2.at[pl.ds((g & 1) * _CHUNK + k, 1)],
                    sem,
                ).wait()
            blk = xv[g >> 3, pl.ds((g & 7) * 16, 16)]
            for k in range(_CHUNK):
                v = blk[k]
                b = (g & 1) * _CHUNK + k
                i_loc = g * _CHUNK + k
                for eb in range(_EMBED // 16):
                    outv[i_loc, pl.ds(eb * 16, 16)] = bands2[b, v & 7, pl.ds(eb * 16, 16)]
            return ()

        lax.fori_loop(0, _NCHUNK, chunk_body, ())
        pltpu.sync_copy(outv, out_hbm.at[pl.ds(base, _BPW)])

    return gather_kernel

_gather = _make_gather()


@jax.jit
def kernel(X, embedding):
    t3 = jnp.reshape(embedding, (125000, 8, _EMBED))
    return _gather(X.astype(jnp.int32), t3)


# speedup vs baseline: 2.5064x; 1.0175x over previous
import functools

import jax
import jax.numpy as jnp
from jax import lax
from jax.experimental import pallas as pl
from jax.experimental.pallas import tpu as pltpu
from jax.experimental.pallas import tpu_sc as plsc

from antenv.accelerators import mock_tpu
from axiom.mock_tpu import make_compilable_single_device_mesh

_BATCH = 16384
_EMBED = 64
_NW = 32
_BPW = _BATCH // _NW
_CHUNK = 32
_NCHUNK = _BPW // _CHUNK


def _make_gather():
    mesh = plsc.VectorSubcoreMesh(core_axis_name="c", subcore_axis_name="s")

    @functools.partial(
        pl.kernel,
        mesh=mesh,
        out_type=jax.ShapeDtypeStruct((_BATCH, _EMBED), jnp.float32),
        scratch_types=[
            pltpu.VMEM((4, 128), jnp.int32),
            pltpu.VMEM((2 * _CHUNK, 8, _EMBED), jnp.float32),
            pltpu.VMEM((2 * _CHUNK, _EMBED), jnp.float32),
            pltpu.SemaphoreType.DMA,
            pltpu.SemaphoreType.DMA,
        ],
        compiler_params=pltpu.CompilerParams(
            use_tc_tiling_on_sc=True, needs_layout_passes=False
        ),
    )
    def gather_kernel(idx_hbm, table_hbm, out_hbm, xv, bands2, stage, sem, osem):

        wid = lax.axis_index("s") * 2 + lax.axis_index("c")
        base = wid * _BPW
        for j in range(4):
            pltpu.sync_copy(idx_hbm.at[pl.ds(base + j * 128, 128)], xv.at[j])

        def enqueue(c, buf):
            blks = [
                xv[c >> 2, pl.ds((c & 3) * _CHUNK + t * 16, 16)]
                for t in range(_CHUNK // 16)
            ]
            for k in range(_CHUNK):
                v = blks[k // 16][k % 16]
                pltpu.async_copy(
                    table_hbm.at[pl.ds(v >> 3, 1)],
                    bands2.at[pl.ds(buf * _CHUNK + k, 1)],
                    sem,
                )

        enqueue(0, 0)

        def chunk_body(g, _):
            nxt = g + 1

            @pl.when(nxt < _NCHUNK)
            def _():
                enqueue(nxt, nxt & 1)

            pltpu.make_async_copy(
                table_hbm.at[pl.ds(0, _CHUNK)],
                bands2.at[pl.ds((g & 1) * _CHUNK, _CHUNK)],
                sem,
            ).wait()

            @pl.when(g >= 2)
            def _():
                pltpu.make_async_copy(
                    stage.at[pl.ds((g & 1) * _CHUNK, _CHUNK)],
                    out_hbm.at[pl.ds(base, _CHUNK)],
                    osem,
                ).wait()

            blks = [
                xv[g >> 2, pl.ds((g & 3) * _CHUNK + t * 16, 16)]
                for t in range(_CHUNK // 16)
            ]
            for k in range(_CHUNK):
                v = blks[k // 16][k % 16]
                b = (g & 1) * _CHUNK + k
                for eb in range(_EMBED // 16):
                    stage[(g & 1) * _CHUNK + k, pl.ds(eb * 16, 16)] = bands2[b, v & 7, pl.ds(eb * 16, 16)]
            pltpu.async_copy(
                stage.at[pl.ds((g & 1) * _CHUNK, _CHUNK)],
                out_hbm.at[pl.ds(base + g * _CHUNK, _CHUNK)],
                osem,
            )
            return ()

        lax.fori_loop(0, _NCHUNK, chunk_body, ())
        for _t in range(2):
            pltpu.make_async_copy(
                stage.at[pl.ds(_t * _CHUNK, _CHUNK)],
                out_hbm.at[pl.ds(base, _CHUNK)],
                osem,
            ).wait()

    return gather_kernel

_gather = _make_gather()


@jax.jit
def kernel(X, embedding):
    t3 = jnp.reshape(embedding, (125000, 8, _EMBED))
    return _gather(X.astype(jnp.int32), t3)
